# baseline (speedup 1.0000x reference)
import jax
import jax.numpy as jnp
from jax import lax
from jax.experimental import pallas as pl
from jax.experimental.pallas import tpu as pltpu
from jax.experimental.pallas import tpu_sc as plsc

EMB = 64
WIN = 5
RAD = WIN // 2
CHUNK = 98
TOKW = CHUNK + WIN - 1
NCORES = 2
NSUB = 16
NWORK = NCORES * NSUB
BLANES = 32
NEB = EMB // BLANES


def _sc_body(tok_hbm, ctr_hbm, w_hbm, k_hbm, out_hbm,
             tok_all, ctr_all,
             w_rows0, k_rows0, out_v0,
             w_rows1, k_rows1, out_v1,
             sem_w0, sem_k0, sem_o0, sem_w1, sem_k1, sem_o1):
    c = lax.axis_index("c")
    s = lax.axis_index("s")
    wid = s * NCORES + c
    per = tok_hbm.shape[0] // NWORK
    base = wid * per

    pltpu.sync_copy(tok_hbm.at[pl.ds(base, per)], tok_all)
    pltpu.sync_copy(ctr_hbm.at[pl.ds(base, per)], ctr_all)

    bufs = ((w_rows0, k_rows0, out_v0, sem_w0, sem_k0, sem_o0),
            (w_rows1, k_rows1, out_v1, sem_w1, sem_k1, sem_o1))

    def issue(j, buf):
        w_rows, k_rows, _, sem_w, sem_k, _ = buf
        jj = jnp.minimum(j, per - 1)
        pltpu.async_copy(w_hbm.at[tok_all.at[jj]], w_rows, sem_w)
        pltpu.async_copy(k_hbm.at[ctr_all.at[jj]], k_rows, sem_k)

    def wait_gathers(buf):
        w_rows, k_rows, _, sem_w, sem_k, _ = buf
        pltpu.make_async_copy(w_hbm.at[tok_all.at[0]], w_rows, sem_w).wait()
        pltpu.make_async_copy(k_hbm.at[ctr_all.at[0]], k_rows, sem_k).wait()

    def wait_out(buf):
        _, _, out_v, _, _, sem_o = buf
        pltpu.make_async_copy(out_v, out_hbm.at[base], sem_o).wait()

    def compute(buf):
        w_rows, k_rows, out_v = buf[0], buf[1], buf[2]

        @pl.loop(0, CHUNK)
        def _(p):
            for eb in range(NEB):
                off = eb * BLANES
                m = None
                for w in range(WIN):
                    a = w_rows[pl.ds(p + w, 1), pl.ds(off, BLANES)]
                    b = k_rows[pl.ds(p, 1), pl.ds(w * EMB + off, BLANES)]
                    prod = a * b
                    m = prod if m is None else jnp.maximum(m, prod)
                out_v[pl.ds(p, 1), pl.ds(off, BLANES)] = m

    issue(0, bufs[0])

    @pl.loop(0, per, step=2)
    def _(i):
        issue(i + 1, bufs[1])
        wait_gathers(bufs[0])

        @pl.when(i > 0)
        def _():
            wait_out(bufs[0])

        compute(bufs[0])
        pltpu.async_copy(out_v0, out_hbm.at[base + i], sem_o0)

        issue(i + 2, bufs[0])
        wait_gathers(bufs[1])

        @pl.when(i > 0)
        def _():
            wait_out(bufs[1])

        compute(bufs[1])
        pltpu.async_copy(out_v1, out_hbm.at[base + i + 1], sem_o1)

    wait_gathers(bufs[0])
    wait_out(bufs[0])
    wait_out(bufs[1])


def _tc_cast_bf16(x):
    n = x.size
    xf = x.reshape(n // 128, 128)
    rows = xf.shape[0]
    blk = rows
    if rows % 8 == 0:
        for cand in range(min(rows, 8192) // 8 * 8, 0, -8):
            if rows % cand == 0:
                blk = cand
                break

    def body(x_ref, o_ref):
        o_ref[...] = x_ref[...].astype(jnp.bfloat16)

    out = pl.pallas_call(
        body,
        grid=(rows // blk,),
        in_specs=[pl.BlockSpec((blk, 128), lambda i: (i, 0))],
        out_specs=pl.BlockSpec((blk, 128), lambda i: (i, 0)),
        out_shape=jax.ShapeDtypeStruct((rows, 128), jnp.bfloat16),
    )(xf)
    return out.reshape(x.shape)


def _tc_cast_f32(x):
    n = x.size
    xf = x.reshape(n // 128, 128)
    rows = xf.shape[0]
    blk = rows
    if rows % 8 == 0:
        for cand in range(min(rows, 8192) // 8 * 8, 0, -8):
            if rows % cand == 0:
                blk = cand
                break

    def body(x_ref, o_ref):
        o_ref[...] = x_ref[...].astype(jnp.float32)

    out = pl.pallas_call(
        body,
        grid=(rows // blk,),
        in_specs=[pl.BlockSpec((blk, 128), lambda i: (i, 0))],
        out_specs=pl.BlockSpec((blk, 128), lambda i: (i, 0)),
        out_shape=jax.ShapeDtypeStruct((rows, 128), jnp.float32),
    )(xf)
    return out.reshape(x.shape)


def kernel(seq, W, K):
    B, L = seq.shape
    vocab = W.shape[0]
    n_pos = L - WIN + 1
    nper = n_pos // CHUNK
    nchunk = B * nper
    seq = seq.astype(jnp.int32)

    tok = jnp.stack(
        [seq[:, j * CHUNK: j * CHUNK + TOKW] for j in range(nper)], axis=1
    ).reshape(nchunk, TOKW)
    ctr = jnp.stack(
        [seq[:, j * CHUNK + RAD: j * CHUNK + RAD + CHUNK] for j in range(nper)],
        axis=1,
    ).reshape(nchunk, CHUNK)
    w16 = _tc_cast_bf16(W)
    k16 = _tc_cast_bf16(K.reshape(vocab, WIN * EMB))

    per = nchunk // NWORK
    mesh = plsc.VectorSubcoreMesh(core_axis_name="c", subcore_axis_name="s")
    fn = pl.kernel(
        _sc_body,
        out_type=jax.ShapeDtypeStruct((nchunk, CHUNK, EMB), jnp.bfloat16),
        mesh=mesh,
        compiler_params=pltpu.CompilerParams(use_tc_tiling_on_sc=False),
        scratch_types=[
            pltpu.VMEM((per, TOKW), jnp.int32),
            pltpu.VMEM((per, CHUNK), jnp.int32),
            pltpu.VMEM((TOKW, EMB), jnp.bfloat16),
            pltpu.VMEM((CHUNK, WIN * EMB), jnp.bfloat16),
            pltpu.VMEM((CHUNK, EMB), jnp.bfloat16),
            pltpu.VMEM((TOKW, EMB), jnp.bfloat16),
            pltpu.VMEM((CHUNK, WIN * EMB), jnp.bfloat16),
            pltpu.VMEM((CHUNK, EMB), jnp.bfloat16),
            pltpu.SemaphoreType.DMA,
            pltpu.SemaphoreType.DMA,
            pltpu.SemaphoreType.DMA,
            pltpu.SemaphoreType.DMA,
            pltpu.SemaphoreType.DMA,
            pltpu.SemaphoreType.DMA,
        ],
    )
    out = fn(tok, ctr, w16, k16)
    return _tc_cast_f32(out).reshape(B, n_pos, EMB)


# speedup vs baseline: 1.2541x; 1.2541x over previous
import jax
import jax.numpy as jnp
from jax import lax
from jax.experimental import pallas as pl
from jax.experimental.pallas import tpu as pltpu
from jax.experimental.pallas import tpu_sc as plsc

EMB = 64
WIN = 5
RAD = WIN // 2
CHUNK = 98
TOKW = CHUNK + WIN - 1
NCORES = 2
NSUB = 16
NWORK = NCORES * NSUB
BLANES = 32
NEB = EMB // BLANES


def _sc_body(tok_hbm, ctr_hbm, w_hbm, k_hbm, out_hbm,
             tok_all, ctr_all,
             w_rows0, k_rows0, out_v0,
             w_rows1, k_rows1, out_v1,
             sem_w0, sem_k0, sem_o0, sem_w1, sem_k1, sem_o1):
    c = lax.axis_index("c")
    s = lax.axis_index("s")
    wid = s * NCORES + c
    per = tok_hbm.shape[0] // NWORK
    base = wid * per

    pltpu.sync_copy(tok_hbm.at[pl.ds(base, per)], tok_all)
    pltpu.sync_copy(ctr_hbm.at[pl.ds(base, per)], ctr_all)

    bufs = ((w_rows0, k_rows0, out_v0, sem_w0, sem_k0, sem_o0),
            (w_rows1, k_rows1, out_v1, sem_w1, sem_k1, sem_o1))

    def issue(j, buf):
        w_rows, k_rows, _, sem_w, sem_k, _ = buf
        jj = jnp.minimum(j, per - 1)
        pltpu.async_copy(w_hbm.at[tok_all.at[jj]], w_rows, sem_w)
        pltpu.async_copy(k_hbm.at[ctr_all.at[jj]], k_rows, sem_k)

    def wait_gathers(buf):
        w_rows, k_rows, _, sem_w, sem_k, _ = buf
        pltpu.make_async_copy(w_hbm.at[tok_all.at[0]], w_rows, sem_w).wait()
        pltpu.make_async_copy(k_hbm.at[ctr_all.at[0]], k_rows, sem_k).wait()

    def wait_out(buf):
        _, _, out_v, _, _, sem_o = buf
        pltpu.make_async_copy(out_v, out_hbm.at[base], sem_o).wait()

    def compute(buf):
        w_rows, k_rows, out_v = buf[0], buf[1], buf[2]

        @pl.loop(0, CHUNK)
        def _(p):
            for eb in range(NEB):
                off = eb * BLANES
                m = None
                for w in range(WIN):
                    a = w_rows[pl.ds(p + w, 1), pl.ds(off, BLANES)]
                    b = k_rows[pl.ds(p, 1), pl.ds(w, 1), pl.ds(off, BLANES)].reshape(1, BLANES)
                    prod = a * b
                    m = prod if m is None else jnp.maximum(m, prod)
                out_v[pl.ds(p, 1), pl.ds(off, BLANES)] = m

    issue(0, bufs[0])

    @pl.loop(0, per, step=2)
    def _(i):
        issue(i + 1, bufs[1])
        wait_gathers(bufs[0])

        @pl.when(i > 0)
        def _():
            wait_out(bufs[0])

        compute(bufs[0])
        pltpu.async_copy(out_v0, out_hbm.at[base + i], sem_o0)

        issue(i + 2, bufs[0])
        wait_gathers(bufs[1])

        @pl.when(i > 0)
        def _():
            wait_out(bufs[1])

        compute(bufs[1])
        pltpu.async_copy(out_v1, out_hbm.at[base + i + 1], sem_o1)

    wait_gathers(bufs[0])
    wait_out(bufs[0])
    wait_out(bufs[1])


def kernel(seq, W, K):
    B, L = seq.shape
    vocab = W.shape[0]
    n_pos = L - WIN + 1
    nper = n_pos // CHUNK
    nchunk = B * nper
    seq = seq.astype(jnp.int32)

    tok = jnp.stack(
        [seq[:, j * CHUNK: j * CHUNK + TOKW] for j in range(nper)], axis=1
    ).reshape(nchunk, TOKW)
    ctr = jnp.stack(
        [seq[:, j * CHUNK + RAD: j * CHUNK + RAD + CHUNK] for j in range(nper)],
        axis=1,
    ).reshape(nchunk, CHUNK)
    w16 = W.astype(jnp.bfloat16)
    k16 = K.astype(jnp.bfloat16)

    per = nchunk // NWORK
    mesh = plsc.VectorSubcoreMesh(core_axis_name="c", subcore_axis_name="s")
    fn = pl.kernel(
        _sc_body,
        out_type=jax.ShapeDtypeStruct((nchunk, CHUNK, EMB), jnp.bfloat16),
        mesh=mesh,
        compiler_params=pltpu.CompilerParams(use_tc_tiling_on_sc=False),
        scratch_types=[
            pltpu.VMEM((per, TOKW), jnp.int32),
            pltpu.VMEM((per, CHUNK), jnp.int32),
            pltpu.VMEM((TOKW, EMB), jnp.bfloat16),
            pltpu.VMEM((CHUNK, WIN, EMB), jnp.bfloat16),
            pltpu.VMEM((CHUNK, EMB), jnp.bfloat16),
            pltpu.VMEM((TOKW, EMB), jnp.bfloat16),
            pltpu.VMEM((CHUNK, WIN, EMB), jnp.bfloat16),
            pltpu.VMEM((CHUNK, EMB), jnp.bfloat16),
            pltpu.SemaphoreType.DMA,
            pltpu.SemaphoreType.DMA,
            pltpu.SemaphoreType.DMA,
            pltpu.SemaphoreType.DMA,
            pltpu.SemaphoreType.DMA,
            pltpu.SemaphoreType.DMA,
        ],
    )
    out = fn(tok, ctr, w16, k16)
    return out.astype(jnp.float32).reshape(B, n_pos, EMB)


# speedup vs baseline: 1.3743x; 1.0958x over previous
import jax
import jax.numpy as jnp
from jax import lax
from jax.experimental import pallas as pl
from jax.experimental.pallas import tpu as pltpu
from jax.experimental.pallas import tpu_sc as plsc

EMB = 64
WIN = 5
RAD = WIN // 2
CHUNK = 98
TOKW = CHUNK + WIN - 1
NCORES = 2
NSUB = 16
NWORK = NCORES * NSUB
BLANES = 32
NEB = EMB // BLANES


def _sc_body(tok_hbm, ctr_hbm, w_hbm, k_hbm, out_hbm,
             tok_all, ctr_all,
             w_rows0, k_rows0, out_v0,
             w_rows1, k_rows1, out_v1,
             sem_w0, sem_k0, sem_o0, sem_w1, sem_k1, sem_o1):
    c = lax.axis_index("c")
    s = lax.axis_index("s")
    wid = s * NCORES + c
    per = tok_hbm.shape[0] // NWORK
    base = wid * per

    pltpu.sync_copy(tok_hbm.at[pl.ds(base, per)], tok_all)
    pltpu.sync_copy(ctr_hbm.at[pl.ds(base, per)], ctr_all)

    bufs = ((w_rows0, k_rows0, out_v0, sem_w0, sem_k0, sem_o0),
            (w_rows1, k_rows1, out_v1, sem_w1, sem_k1, sem_o1))

    def issue(j, buf):
        w_rows, k_rows, _, sem_w, sem_k, _ = buf
        jj = jnp.minimum(j, per - 1)
        pltpu.async_copy(w_hbm.at[tok_all.at[jj]], w_rows, sem_w)
        pltpu.async_copy(k_hbm.at[ctr_all.at[jj]], k_rows, sem_k)

    def wait_gathers(buf):
        w_rows, k_rows, _, sem_w, sem_k, _ = buf
        pltpu.make_async_copy(w_hbm.at[tok_all.at[0]], w_rows, sem_w).wait()
        pltpu.make_async_copy(k_hbm.at[ctr_all.at[0]], k_rows, sem_k).wait()

    def wait_out(buf):
        _, _, out_v, _, _, sem_o = buf
        pltpu.make_async_copy(out_v, out_hbm.at[base], sem_o).wait()

    def compute(buf):
        w_rows, k_rows, out_v = buf[0], buf[1], buf[2]

        @pl.loop(0, CHUNK)
        def _(p):
            for eb in range(NEB):
                off = eb * BLANES
                m = None
                for w in range(WIN):
                    a = w_rows[p + w, pl.ds(off, BLANES)]
                    b = k_rows[p, pl.ds(w * EMB + off, BLANES)]
                    prod = a * b
                    m = prod if m is None else jnp.maximum(m, prod)
                out_v[p, pl.ds(off, BLANES)] = m

    issue(0, bufs[0])

    @pl.loop(0, per, step=2)
    def _(i):
        issue(i + 1, bufs[1])
        wait_gathers(bufs[0])

        @pl.when(i > 0)
        def _():
            wait_out(bufs[0])

        compute(bufs[0])
        pltpu.async_copy(out_v0, out_hbm.at[base + i], sem_o0)

        issue(i + 2, bufs[0])
        wait_gathers(bufs[1])

        @pl.when(i > 0)
        def _():
            wait_out(bufs[1])

        compute(bufs[1])
        pltpu.async_copy(out_v1, out_hbm.at[base + i + 1], sem_o1)

    wait_gathers(bufs[0])
    wait_out(bufs[0])
    wait_out(bufs[1])


def kernel(seq, W, K):
    B, L = seq.shape
    vocab = W.shape[0]
    n_pos = L - WIN + 1
    nper = n_pos // CHUNK
    nchunk = B * nper
    seq = seq.astype(jnp.int32)

    tok = jnp.stack(
        [seq[:, j * CHUNK: j * CHUNK + TOKW] for j in range(nper)], axis=1
    ).reshape(nchunk, TOKW)
    ctr = jnp.stack(
        [seq[:, j * CHUNK + RAD: j * CHUNK + RAD + CHUNK] for j in range(nper)],
        axis=1,
    ).reshape(nchunk, CHUNK)
    w16 = W.astype(jnp.bfloat16)
    k16 = K.reshape(vocab, WIN * EMB).astype(jnp.bfloat16)

    per = nchunk // NWORK
    mesh = plsc.VectorSubcoreMesh(core_axis_name="c", subcore_axis_name="s")
    fn = pl.kernel(
        _sc_body,
        out_type=jax.ShapeDtypeStruct((nchunk, CHUNK, EMB), jnp.bfloat16),
        mesh=mesh,
        compiler_params=pltpu.CompilerParams(use_tc_tiling_on_sc=False),
        scratch_types=[
            pltpu.VMEM((per, TOKW), jnp.int32),
            pltpu.VMEM((per, CHUNK), jnp.int32),
            pltpu.VMEM((TOKW, EMB), jnp.bfloat16),
            pltpu.VMEM((CHUNK, WIN * EMB), jnp.bfloat16),
            pltpu.VMEM((CHUNK, EMB), jnp.bfloat16),
            pltpu.VMEM((TOKW, EMB), jnp.bfloat16),
            pltpu.VMEM((CHUNK, WIN * EMB), jnp.bfloat16),
            pltpu.VMEM((CHUNK, EMB), jnp.bfloat16),
            pltpu.SemaphoreType.DMA,
            pltpu.SemaphoreType.DMA,
            pltpu.SemaphoreType.DMA,
            pltpu.SemaphoreType.DMA,
            pltpu.SemaphoreType.DMA,
            pltpu.SemaphoreType.DMA,
        ],
    )
    out = fn(tok, ctr, w16, k16)
    return out.reshape(B, n_pos, EMB).astype(jnp.float32)


# speedup vs baseline: 1.3747x; 1.0003x over previous
import jax
import jax.numpy as jnp
from jax import lax
from jax.experimental import pallas as pl
from jax.experimental.pallas import tpu as pltpu
from jax.experimental.pallas import tpu_sc as plsc

EMB = 64
WIN = 5
RAD = WIN // 2
CHUNK = 98
TOKW = CHUNK + WIN - 1
NCORES = 2
NSUB = 16
NWORK = NCORES * NSUB
BLANES = 32
NEB = EMB // BLANES


def _sc_body(tok_hbm, ctr_hbm, w_hbm, k_hbm, out_hbm,
             tok_all, ctr_all,
             w_rows0, k_rows0, out_v0,
             w_rows1, k_rows1, out_v1,
             sem_w0, sem_k0, sem_o0, sem_w1, sem_k1, sem_o1):
    c = lax.axis_index("c")
    s = lax.axis_index("s")
    wid = s * NCORES + c
    per = tok_hbm.shape[0] // NWORK
    base = wid * per

    pltpu.sync_copy(tok_hbm.at[pl.ds(base, per)], tok_all)
    pltpu.sync_copy(ctr_hbm.at[pl.ds(base, per)], ctr_all)

    bufs = ((w_rows0, k_rows0, out_v0, sem_w0, sem_k0, sem_o0),
            (w_rows1, k_rows1, out_v1, sem_w1, sem_k1, sem_o1))

    def issue(j, buf):
        w_rows, k_rows, _, sem_w, sem_k, _ = buf
        jj = jnp.minimum(j, per - 1)
        pltpu.async_copy(w_hbm.at[tok_all.at[jj]], w_rows, sem_w)
        pltpu.async_copy(k_hbm.at[ctr_all.at[jj]], k_rows, sem_k)

    def wait_gathers(buf):
        w_rows, k_rows, _, sem_w, sem_k, _ = buf
        pltpu.make_async_copy(w_hbm.at[tok_all.at[0]], w_rows, sem_w).wait()
        pltpu.make_async_copy(k_hbm.at[ctr_all.at[0]], k_rows, sem_k).wait()

    def wait_out(buf):
        _, _, out_v, _, _, sem_o = buf
        pltpu.make_async_copy(out_v, out_hbm.at[base], sem_o).wait()

    def compute(buf):
        w_rows, k_rows, out_v = buf[0], buf[1], buf[2]

        @pl.loop(0, CHUNK)
        def _(p):
            for eb in range(NEB):
                off = eb * BLANES
                m = None
                for w in range(WIN):
                    a = w_rows[p + w, pl.ds(off, BLANES)]
                    b = k_rows[p, pl.ds(w * EMB + off, BLANES)]
                    prod = a * b
                    m = prod if m is None else jnp.maximum(m, prod)
                out_v[p, pl.ds(off, BLANES)] = m

    issue(0, bufs[0])

    @pl.loop(0, per, step=2)
    def _(i):
        issue(i + 1, bufs[1])
        wait_gathers(bufs[0])

        @pl.when(i > 0)
        def _():
            wait_out(bufs[0])

        compute(bufs[0])
        pltpu.async_copy(out_v0, out_hbm.at[base + i], sem_o0)

        issue(i + 2, bufs[0])
        wait_gathers(bufs[1])

        @pl.when(i > 0)
        def _():
            wait_out(bufs[1])

        compute(bufs[1])
        pltpu.async_copy(out_v1, out_hbm.at[base + i + 1], sem_o1)

    wait_gathers(bufs[0])
    wait_out(bufs[0])
    wait_out(bufs[1])


def kernel(seq, W, K):
    B, L = seq.shape
    vocab = W.shape[0]
    n_pos = L - WIN + 1
    nper = n_pos // CHUNK
    nchunk = B * nper
    seq = seq.astype(jnp.int32)

    tok = jnp.stack(
        [seq[:, j * CHUNK: j * CHUNK + TOKW] for j in range(nper)], axis=1
    ).reshape(nchunk, TOKW)
    ctr = jnp.stack(
        [seq[:, j * CHUNK + RAD: j * CHUNK + RAD + CHUNK] for j in range(nper)],
        axis=1,
    ).reshape(nchunk, CHUNK)
    w16 = W.astype(jnp.bfloat16)
    k16 = K.astype(jnp.bfloat16).reshape(vocab, WIN * EMB)

    per = nchunk // NWORK
    mesh = plsc.VectorSubcoreMesh(core_axis_name="c", subcore_axis_name="s")
    fn = pl.kernel(
        _sc_body,
        out_type=jax.ShapeDtypeStruct((nchunk, CHUNK, EMB), jnp.bfloat16),
        mesh=mesh,
        compiler_params=pltpu.CompilerParams(use_tc_tiling_on_sc=False),
        scratch_types=[
            pltpu.VMEM((per, TOKW), jnp.int32),
            pltpu.VMEM((per, CHUNK), jnp.int32),
            pltpu.VMEM((TOKW, EMB), jnp.bfloat16),
            pltpu.VMEM((CHUNK, WIN * EMB), jnp.bfloat16),
            pltpu.VMEM((CHUNK, EMB), jnp.bfloat16),
            pltpu.VMEM((TOKW, EMB), jnp.bfloat16),
            pltpu.VMEM((CHUNK, WIN * EMB), jnp.bfloat16),
            pltpu.VMEM((CHUNK, EMB), jnp.bfloat16),
            pltpu.SemaphoreType.DMA,
            pltpu.SemaphoreType.DMA,
            pltpu.SemaphoreType.DMA,
            pltpu.SemaphoreType.DMA,
            pltpu.SemaphoreType.DMA,
            pltpu.SemaphoreType.DMA,
        ],
    )
    out = fn(tok, ctr, w16, k16)
    return out.reshape(B, n_pos, EMB).astype(jnp.float32)
